# baseline (speedup 1.0000x reference)
import functools
import math

import jax
import jax.numpy as jnp
import numpy as np
from jax import lax
from jax.experimental import pallas as pl
from jax.experimental.pallas import tpu as pltpu
from jax.experimental.pallas import tpu_sc as plsc

VOCAB = 100000
D_MODEL = 768
PE_LEN = 2048
SCALE = math.sqrt(float(D_MODEL))

_INFO = plsc.get_sparse_core_info()
NC = _INFO.num_cores
NS = _INFO.num_subcores
LANES = _INFO.num_lanes
NW = NC * NS


def _positional_encoding(length, depth):
    half = depth / 2
    positions = np.arange(length)[:, np.newaxis]
    depths = np.arange(half)[np.newaxis, :] / half
    angle_rates = 1.0 / (10000.0 ** depths)
    angle_rads = positions * angle_rates
    pe = np.concatenate([np.sin(angle_rads), np.cos(angle_rads)], axis=-1)
    return jnp.asarray(pe, dtype=jnp.float32)


@functools.partial(jax.jit, static_argnames=("batch", "seq"))
def _lookup(x_flat, table, pe, *, batch, seq):
    total = batch * seq
    assert total % NW == 0
    b_per_w = total // NW
    ch = 64
    nchunk = b_per_w // ch
    assert seq % b_per_w == 0

    mesh = plsc.VectorSubcoreMesh(core_axis_name="c", subcore_axis_name="s")

    @functools.partial(
        pl.kernel,
        mesh=mesh,
        out_type=jax.ShapeDtypeStruct((total, D_MODEL), jnp.float32),
        scratch_types=[
            pltpu.VMEM((ch,), jnp.int32),
            pltpu.VMEM((ch, D_MODEL), jnp.float32),
            pltpu.VMEM((ch, D_MODEL), jnp.float32),
            pltpu.SemaphoreType.DMA,
        ],
    )
    def k(x_hbm, pe_hbm, table_hbm, out_hbm, idx_v, rows_v, pe_v, sem):
        wid = lax.axis_index("s") * NC + lax.axis_index("c")
        r0 = wid * b_per_w
        t0 = lax.rem(r0, seq)

        def chunk(c, carry):
            rbase = r0 + c * ch
            tbase = t0 + c * ch
            pltpu.sync_copy(x_hbm.at[pl.ds(rbase, ch)], idx_v)
            gather = pltpu.async_copy(table_hbm.at[idx_v], rows_v, sem)
            pltpu.sync_copy(pe_hbm.at[pl.ds(tbase, ch)], pe_v)
            gather.wait()

            def row(i, carry2):
                def col(j, carry3):
                    sl = pl.ds(j * LANES, LANES)
                    rows_v[i, sl] = rows_v[i, sl] * SCALE + pe_v[i, sl]
                    return carry3
                return lax.fori_loop(0, D_MODEL // LANES, col, carry2)

            lax.fori_loop(0, ch, row, carry)
            pltpu.sync_copy(rows_v, out_hbm.at[pl.ds(rbase, ch)])
            return carry

        lax.fori_loop(0, nchunk, chunk, 0)

    return k(x_flat, pe, table)


def kernel(x, table):
    batch, seq = x.shape
    pe = _positional_encoding(PE_LEN, D_MODEL)[:seq]
    x_flat = x.reshape(-1).astype(jnp.int32)
    out = _lookup(x_flat, table, pe, batch=batch, seq=seq)
    return out.reshape(batch, seq, D_MODEL)


# speedup vs baseline: 1.6828x; 1.6828x over previous
import functools
import math

import jax
import jax.numpy as jnp
import numpy as np
from jax import lax
from jax.experimental import pallas as pl
from jax.experimental.pallas import tpu as pltpu
from jax.experimental.pallas import tpu_sc as plsc

VOCAB = 100000
D_MODEL = 768
PE_LEN = 2048
SCALE = math.sqrt(float(D_MODEL))

_INFO = plsc.get_sparse_core_info()
NC = _INFO.num_cores
NS = _INFO.num_subcores
LANES = _INFO.num_lanes
NW = NC * NS

CH = 16
NBUF = 4


def _positional_encoding(length, depth):
    half = depth / 2
    positions = np.arange(length)[:, np.newaxis]
    depths = np.arange(half)[np.newaxis, :] / half
    angle_rates = 1.0 / (10000.0 ** depths)
    angle_rads = positions * angle_rates
    pe = np.concatenate([np.sin(angle_rads), np.cos(angle_rads)], axis=-1)
    return pe


@functools.partial(jax.jit, static_argnames=("batch", "seq"))
def _lookup(x_flat, table, pe, *, batch, seq):
    total = batch * seq
    assert total % NW == 0
    b_per_w = total // NW
    nchunk = b_per_w // CH
    assert seq % b_per_w == 0

    mesh = plsc.VectorSubcoreMesh(core_axis_name="c", subcore_axis_name="s")

    @functools.partial(
        pl.kernel,
        mesh=mesh,
        out_type=jax.ShapeDtypeStruct((total, D_MODEL), jnp.float32),
        scratch_types=[
            pltpu.VMEM((NBUF, CH), jnp.int32),
            pltpu.VMEM((NBUF, CH, D_MODEL), jnp.float32),
            pltpu.VMEM((NBUF, CH, D_MODEL), jnp.float32),
            [pltpu.SemaphoreType.DMA] * NBUF,
            [pltpu.SemaphoreType.DMA] * NBUF,
            [pltpu.SemaphoreType.DMA] * NBUF,
        ],
    )
    def k(x_hbm, pe_hbm, table_hbm, out_hbm, idx_v, acc_v, emb_v, pe_sems,
          g_sems, o_sems):
        wid = lax.axis_index("s") * NC + lax.axis_index("c")
        r0 = wid * b_per_w
        t0 = lax.rem(r0, seq)

        pe_cps = [None] * NBUF
        g_cps = [None] * NBUF
        out_cps = [None] * NBUF

        def prep(c):
            p = c % NBUF
            if out_cps[p] is not None:
                out_cps[p].wait()
            rbase = r0 + c * CH
            pltpu.sync_copy(x_hbm.at[pl.ds(rbase, CH)], idx_v.at[p])
            pe_cps[p] = pltpu.async_copy(
                pe_hbm.at[pl.ds(t0 + c * CH, CH)], acc_v.at[p], pe_sems[p]
            )
            g_cps[p] = pltpu.async_copy(
                table_hbm.at[idx_v.at[p]], emb_v.at[p], g_sems[p]
            )

        for c in range(NBUF - 1):
            prep(c)

        for c in range(nchunk):
            p = c % NBUF
            g_cps[p].wait()
            pe_cps[p].wait()

            def row(i, carry):
                for j in range(D_MODEL // LANES):
                    sl = pl.ds(j * LANES, LANES)
                    plsc.addupdate(
                        acc_v.at[p, i, sl], emb_v[p, i, sl] * SCALE
                    )
                return carry

            lax.fori_loop(0, CH, row, 0)
            out_cps[p] = pltpu.async_copy(
                acc_v.at[p], out_hbm.at[pl.ds(r0 + c * CH, CH)], o_sems[p]
            )
            if c + NBUF - 1 < nchunk:
                prep(c + NBUF - 1)

        for p in range(NBUF):
            if out_cps[p] is not None:
                out_cps[p].wait()

    return k(x_flat, pe, table)


def kernel(x, table):
    batch, seq = x.shape
    pe = jnp.asarray(
        _positional_encoding(PE_LEN, D_MODEL)[:seq], dtype=jnp.float32
    )
    x_flat = x.reshape(-1).astype(jnp.int32)
    out = _lookup(x_flat, table, pe, batch=batch, seq=seq)
    return out.reshape(batch, seq, D_MODEL)


# speedup vs baseline: 1.8009x; 1.0702x over previous
import functools
import math

import jax
import jax.numpy as jnp
import numpy as np
from jax import lax
from jax.experimental import pallas as pl
from jax.experimental.pallas import tpu as pltpu
from jax.experimental.pallas import tpu_sc as plsc

VOCAB = 100000
D_MODEL = 768
PE_LEN = 2048
SCALE = math.sqrt(float(D_MODEL))

_INFO = plsc.get_sparse_core_info()
NC = _INFO.num_cores
NS = _INFO.num_subcores
LANES = _INFO.num_lanes
NW = NC * NS

CH = 16
NBUF = 4


def _positional_encoding(length, depth):
    half = depth / 2
    positions = np.arange(length)[:, np.newaxis]
    depths = np.arange(half)[np.newaxis, :] / half
    angle_rates = 1.0 / (10000.0 ** depths)
    angle_rads = positions * angle_rates
    pe = np.concatenate([np.sin(angle_rads), np.cos(angle_rads)], axis=-1)
    return pe


@functools.partial(jax.jit, static_argnames=("batch", "seq"))
def _lookup(x_flat, table, pe, *, batch, seq):
    total = batch * seq
    assert total % NW == 0
    b_per_w = total // NW
    nchunk = b_per_w // CH
    assert seq % b_per_w == 0

    mesh = plsc.VectorSubcoreMesh(core_axis_name="c", subcore_axis_name="s")

    @functools.partial(
        pl.kernel,
        mesh=mesh,
        out_type=jax.ShapeDtypeStruct((total, D_MODEL), jnp.float32),
        scratch_types=[
            pltpu.VMEM((b_per_w,), jnp.int32),
            pltpu.VMEM((NBUF, CH, D_MODEL), jnp.float32),
            pltpu.VMEM((NBUF, CH, D_MODEL), jnp.float32),
            [pltpu.SemaphoreType.DMA] * NBUF,
            [pltpu.SemaphoreType.DMA] * NBUF,
            [pltpu.SemaphoreType.DMA] * NBUF,
        ],
    )
    def k(x_hbm, pe_hbm, table_hbm, out_hbm, idx_v, acc_v, emb_v, pe_sems,
          g_sems, o_sems):
        wid = lax.axis_index("s") * NC + lax.axis_index("c")
        r0 = wid * b_per_w
        t0 = lax.rem(r0, seq)

        pltpu.sync_copy(x_hbm.at[pl.ds(r0, b_per_w)], idx_v)

        pe_cps = [None] * NBUF
        g_cps = [None] * NBUF
        out_cps = [None] * NBUF

        def prep(c):
            p = c % NBUF
            if out_cps[p] is not None:
                out_cps[p].wait()
            pe_cps[p] = pltpu.async_copy(
                pe_hbm.at[pl.ds(t0 + c * CH, CH)], acc_v.at[p], pe_sems[p]
            )
            g_cps[p] = pltpu.async_copy(
                table_hbm.at[idx_v.at[pl.ds(c * CH, CH)]], emb_v.at[p],
                g_sems[p]
            )

        for c in range(NBUF - 1):
            prep(c)

        for c in range(nchunk):
            p = c % NBUF
            g_cps[p].wait()
            pe_cps[p].wait()

            def row(i, carry):
                for j in range(D_MODEL // LANES):
                    sl = pl.ds(j * LANES, LANES)
                    plsc.addupdate(
                        acc_v.at[p, i, sl], emb_v[p, i, sl] * SCALE
                    )
                return carry

            lax.fori_loop(0, CH, row, 0)
            out_cps[p] = pltpu.async_copy(
                acc_v.at[p], out_hbm.at[pl.ds(r0 + c * CH, CH)], o_sems[p]
            )
            if c + NBUF - 1 < nchunk:
                prep(c + NBUF - 1)

        for p in range(NBUF):
            if out_cps[p] is not None:
                out_cps[p].wait()

    return k(x_flat, pe, table)


def kernel(x, table):
    batch, seq = x.shape
    pe = jnp.asarray(
        _positional_encoding(PE_LEN, D_MODEL)[:seq], dtype=jnp.float32
    )
    x_flat = x.reshape(-1).astype(jnp.int32)
    out = _lookup(x_flat, table, pe, batch=batch, seq=seq)
    return out.reshape(batch, seq, D_MODEL)


# speedup vs baseline: 2.3595x; 1.3102x over previous
import functools
import math

import jax
import jax.numpy as jnp
import numpy as np
from jax import lax
from jax.experimental import pallas as pl
from jax.experimental.pallas import tpu as pltpu
from jax.experimental.pallas import tpu_sc as plsc

VOCAB = 100000
D_MODEL = 768
PE_LEN = 2048
SCALE = math.sqrt(float(D_MODEL))

_INFO = plsc.get_sparse_core_info()
NC = _INFO.num_cores
NS = _INFO.num_subcores
LANES = _INFO.num_lanes
NW = NC * NS

CH = 8
NBUF = 3


def _positional_encoding(length, depth):
    half = depth / 2
    positions = np.arange(length)[:, np.newaxis]
    depths = np.arange(half)[np.newaxis, :] / half
    angle_rates = 1.0 / (10000.0 ** depths)
    angle_rads = positions * angle_rates
    pe = np.concatenate([np.sin(angle_rads), np.cos(angle_rads)], axis=-1)
    return pe


@functools.partial(jax.jit, static_argnames=("batch", "seq"))
def _lookup(x, table, pe, *, batch, seq):
    assert seq % NW == 0
    t_per_w = seq // NW
    nchunk = t_per_w // CH

    mesh = plsc.VectorSubcoreMesh(core_axis_name="c", subcore_axis_name="s")

    @functools.partial(
        pl.kernel,
        mesh=mesh,
        out_type=jax.ShapeDtypeStruct((batch, seq, D_MODEL), jnp.float32),
        scratch_types=[
            pltpu.VMEM((batch, t_per_w), jnp.int32),
            pltpu.VMEM((t_per_w, D_MODEL), jnp.float32),
            pltpu.VMEM((NBUF, batch, CH, D_MODEL), jnp.float32),
            pltpu.SemaphoreType.DMA,
            [pltpu.SemaphoreType.DMA] * batch,
            [[pltpu.SemaphoreType.DMA] * batch for _ in range(NBUF)],
            [[pltpu.SemaphoreType.DMA] * batch for _ in range(NBUF)],
        ],
    )
    def k(x_hbm, pe_hbm, table_hbm, out_hbm, idx_v, pe_v, emb_v, pe_sem,
          idx_sems, g_sems, o_sems):
        wid = lax.axis_index("s") * NC + lax.axis_index("c")
        t0 = wid * t_per_w

        idx_cps = [
            pltpu.async_copy(
                x_hbm.at[b, pl.ds(t0, t_per_w)], idx_v.at[b], idx_sems[b]
            )
            for b in range(batch)
        ]
        pe_cp = pltpu.async_copy(pe_hbm.at[pl.ds(t0, t_per_w)], pe_v, pe_sem)
        for cp in idx_cps:
            cp.wait()

        g_cps = [[None] * batch for _ in range(NBUF)]
        out_cps = [None] * NBUF

        def prep(c):
            p = c % NBUF
            if out_cps[p] is not None:
                for cp in out_cps[p]:
                    cp.wait()
            for b in range(batch):
                g_cps[p][b] = pltpu.async_copy(
                    table_hbm.at[idx_v.at[b, pl.ds(c * CH, CH)]],
                    emb_v.at[p, b], g_sems[p][b]
                )

        for c in range(NBUF - 1):
            prep(c)
        pe_cp.wait()

        for c in range(nchunk):
            p = c % NBUF
            for b in range(batch):
                g_cps[p][b].wait()

            def row(i, carry):
                for j in range(D_MODEL // LANES):
                    sl = pl.ds(j * LANES, LANES)
                    pe_reg = pe_v[c * CH + i, sl]
                    for b in range(batch):
                        emb_v[p, b, i, sl] = (
                            emb_v[p, b, i, sl] * SCALE + pe_reg
                        )
                return carry

            lax.fori_loop(0, CH, row, 0)
            out_cps[p] = [
                pltpu.async_copy(
                    emb_v.at[p, b],
                    out_hbm.at[b, pl.ds(t0 + c * CH, CH)], o_sems[p][b]
                )
                for b in range(batch)
            ]
            if c + NBUF - 1 < nchunk:
                prep(c + NBUF - 1)

        for p in range(NBUF):
            if out_cps[p] is not None:
                for cp in out_cps[p]:
                    cp.wait()

    return k(x, pe, table)


def kernel(x, table):
    batch, seq = x.shape
    pe = jnp.asarray(
        _positional_encoding(PE_LEN, D_MODEL)[:seq], dtype=jnp.float32
    )
    return _lookup(x.astype(jnp.int32), table, pe, batch=batch, seq=seq)
